# baseline (speedup 1.0000x reference)
import functools

import jax
import jax.numpy as jnp
from jax import lax
from jax.experimental import pallas as pl
from jax.experimental.pallas import tpu as pltpu
from jax.experimental.pallas import tpu_sc as plsc

B, D = 128, 8192
H = D // 4
MIN_S, MAX_S = 0.05, 0.3
KBLK = 1024
NSTEPS = D // KBLK

NC, NS, L = 2, 16, 16
NW = NC * NS
RPW = B // NW


def _head_kernel(x_ref, w1_ref, b1_ref, w2_ref, b2_ref,
                 s_ref, kexp_ref, acc_ref):
    j = pl.program_id(0)

    @pl.when(j == 0)
    def _init():
        acc_ref[...] = jnp.zeros_like(acc_ref)

    acc_ref[...] += jnp.dot(x_ref[...], w1_ref[...],
                            preferred_element_type=jnp.float32)

    @pl.when(j == NSTEPS - 1)
    def _finish():
        h = jnp.maximum(acc_ref[...] + b1_ref[...], 0.0)
        t = jnp.dot(h, w2_ref[...], preferred_element_type=jnp.float32)
        s = jax.nn.sigmoid(t + b2_ref[...])
        sparsity = MIN_S + (MAX_S - MIN_S) * s
        k = jnp.clip(jnp.round(D * (1.0 - sparsity)), 1.0, float(D))
        s_ref[...] = sparsity
        kexp_ref[...] = jnp.broadcast_to(k, (B, L))


def _head(x, W1, b1, W2, b2):
    return pl.pallas_call(
        _head_kernel,
        grid=(NSTEPS,),
        in_specs=[
            pl.BlockSpec((B, KBLK), lambda j: (0, j)),
            pl.BlockSpec((KBLK, H), lambda j: (j, 0)),
            pl.BlockSpec((1, H), lambda j: (0, 0)),
            pl.BlockSpec((H, 1), lambda j: (0, 0)),
            pl.BlockSpec((1, 1), lambda j: (0, 0)),
        ],
        out_specs=(
            pl.BlockSpec((B, 1), lambda j: (0, 0)),
            pl.BlockSpec((B, L), lambda j: (0, 0)),
        ),
        out_shape=(
            jax.ShapeDtypeStruct((B, 1), jnp.float32),
            jax.ShapeDtypeStruct((B, L), jnp.float32),
        ),
        scratch_shapes=[pltpu.VMEM((B, H), jnp.float32)],
    )(x, W1, b1.reshape(1, H), W2, b2.reshape(1, 1))


_mesh = plsc.VectorSubcoreMesh(core_axis_name="c", subcore_axis_name="s")


@functools.partial(
    pl.kernel,
    mesh=_mesh,
    out_type=(
        jax.ShapeDtypeStruct((B, D), jnp.float32),
        jax.ShapeDtypeStruct((B, D), jnp.float32),
        jax.ShapeDtypeStruct((B * L,), jnp.float32),
        jax.ShapeDtypeStruct((NW * L,), jnp.float32),
    ),
    scratch_types=[
        pltpu.VMEM((RPW * D,), jnp.float32),
        pltpu.VMEM((RPW * D,), jnp.float32),
        pltpu.VMEM((RPW * L,), jnp.float32),
        pltpu.VMEM((RPW * L,), jnp.float32),
        pltpu.VMEM((L,), jnp.float32),
        pltpu.SemaphoreType.DMA,
    ],
)
def _sc_select(x_hbm, kexp_hbm, sparse_hbm, mask_hbm, nnz_hbm, l1_hbm,
               xv, mv, kv, nnzv, l1v, sem):
    wid = lax.axis_index("s") * NC + lax.axis_index("c")
    base = wid * RPW

    copies = []
    for r in range(RPW):
        copies.append(pltpu.async_copy(
            x_hbm.at[base + r], xv.at[pl.ds(r * D, D)], sem))
        copies.append(pltpu.async_copy(
            kexp_hbm.at[pl.ds((base + r) * L, L)], kv.at[pl.ds(r * L, L)], sem))
    for c in copies:
        c.wait()

    iota = lax.iota(jnp.int32, L)
    dnums = lax.GatherDimensionNumbers(
        offset_dims=(), collapsed_slice_dims=(0,), start_index_map=(0,))

    def lane_sum(v):
        for sh in (8, 4, 2, 1):
            v = v + lax.gather(v, (iota ^ sh)[:, None], dnums,
                               slice_sizes=(1,),
                               mode=lax.GatherScatterMode.PROMISE_IN_BOUNDS)
        return v

    ones = jnp.ones((L,), jnp.float32)
    zero_f = jnp.zeros((L,), jnp.float32)
    signmask = jnp.full((L,), 0x7FFFFFFF, jnp.int32)

    l1_acc = zero_f
    for r in range(RPW):
        rb = r * D
        k_vec = kv[pl.ds(r * L, L)]

        def step(i, p, rb=rb):
            bitval = lax.shift_left(jnp.int32(1), jnp.int32(30) - i)
            c = p | jnp.broadcast_to(bitval, (L,))

            def inner(j, acc, rb=rb, c=c):
                v = xv[pl.ds(rb + j * L, L)]
                b = lax.bitcast_convert_type(v, jnp.int32) & signmask
                return acc + jnp.where(b < c, ones, zero_f)

            acc = lax.fori_loop(0, D // L, inner, zero_f, unroll=8)
            return jnp.where(lane_sum(acc) < k_vec, c, p)

        p = lax.fori_loop(0, 31, step, jnp.zeros((L,), jnp.int32))

        def mpass(j, carry, rb=rb, p=p):
            nnz, l1 = carry
            v = xv[pl.ds(rb + j * L, L)]
            b = lax.bitcast_convert_type(v, jnp.int32) & signmask
            mf = jnp.where(b > p, 1.0, 0.0)
            sx = v * mf
            xv[pl.ds(rb + j * L, L)] = sx
            mv[pl.ds(rb + j * L, L)] = mf
            return (nnz + mf, l1 + jnp.abs(sx))

        nnz, l1row = lax.fori_loop(0, D // L, mpass, (zero_f, zero_f),
                                   unroll=4)
        nnzv[pl.ds(r * L, L)] = lane_sum(nnz) * (1.0 / D)
        l1_acc = l1_acc + l1row

    l1v[...] = lane_sum(l1_acc)

    outs = []
    for r in range(RPW):
        outs.append(pltpu.async_copy(
            xv.at[pl.ds(r * D, D)], sparse_hbm.at[base + r], sem))
        outs.append(pltpu.async_copy(
            mv.at[pl.ds(r * D, D)], mask_hbm.at[base + r], sem))
    outs.append(pltpu.async_copy(
        nnzv, nnz_hbm.at[pl.ds(base * L, RPW * L)], sem))
    outs.append(pltpu.async_copy(
        l1v, l1_hbm.at[pl.ds(wid * L, L)], sem))
    for c in outs:
        c.wait()


@jax.jit
def kernel(x, W1, b1, W2, b2):
    sparsity, kexp = _head(x, W1, b1, W2, b2)
    sparse_x, mask, nnz, l1p = _sc_select(x, kexp.reshape(B * L))
    actual_sparsity = nnz.reshape(B, L)[:, 0]
    l1_reg = l1p.reshape(NW, L)[:, 0].sum() * (1.0 / B)
    return (sparse_x, mask, sparsity, actual_sparsity, l1_reg)


# speedup vs baseline: 1.4371x; 1.4371x over previous
import functools

import jax
import jax.numpy as jnp
from jax import lax
from jax.experimental import pallas as pl
from jax.experimental.pallas import tpu as pltpu
from jax.experimental.pallas import tpu_sc as plsc

B, D = 128, 8192
H = D // 4
MIN_S, MAX_S = 0.05, 0.3
KBLK = 1024
NSTEPS = D // KBLK

NC, NS, L = 2, 16, 16
NW = NC * NS
RPW = B // NW


def _head_kernel(x_ref, w1_ref, b1_ref, w2_ref, b2_ref,
                 s_ref, kexp_ref, acc_ref):
    j = pl.program_id(0)

    @pl.when(j == 0)
    def _init():
        acc_ref[...] = jnp.zeros_like(acc_ref)

    acc_ref[...] += jnp.dot(x_ref[...], w1_ref[...],
                            preferred_element_type=jnp.float32)

    @pl.when(j == NSTEPS - 1)
    def _finish():
        h = jnp.maximum(acc_ref[...] + b1_ref[...], 0.0)
        t = jnp.dot(h, w2_ref[...], preferred_element_type=jnp.float32)
        s = jax.nn.sigmoid(t + b2_ref[...])
        sparsity = MIN_S + (MAX_S - MIN_S) * s
        k = jnp.clip(jnp.round(D * (1.0 - sparsity)), 1.0, float(D))
        s_ref[...] = sparsity
        kexp_ref[...] = jnp.broadcast_to(k, (B, L))


def _head(x, W1, b1, W2, b2):
    return pl.pallas_call(
        _head_kernel,
        grid=(NSTEPS,),
        in_specs=[
            pl.BlockSpec((B, KBLK), lambda j: (0, j)),
            pl.BlockSpec((KBLK, H), lambda j: (j, 0)),
            pl.BlockSpec((1, H), lambda j: (0, 0)),
            pl.BlockSpec((H, 1), lambda j: (0, 0)),
            pl.BlockSpec((1, 1), lambda j: (0, 0)),
        ],
        out_specs=(
            pl.BlockSpec((B, 1), lambda j: (0, 0)),
            pl.BlockSpec((B, L), lambda j: (0, 0)),
        ),
        out_shape=(
            jax.ShapeDtypeStruct((B, 1), jnp.float32),
            jax.ShapeDtypeStruct((B, L), jnp.float32),
        ),
        scratch_shapes=[pltpu.VMEM((B, H), jnp.float32)],
    )(x, W1, b1.reshape(1, H), W2, b2.reshape(1, 1))


_mesh = plsc.VectorSubcoreMesh(core_axis_name="c", subcore_axis_name="s")


@functools.partial(
    pl.kernel,
    mesh=_mesh,
    out_type=(
        jax.ShapeDtypeStruct((B, D), jnp.float32),
        jax.ShapeDtypeStruct((B, D), jnp.float32),
        jax.ShapeDtypeStruct((B * L,), jnp.float32),
        jax.ShapeDtypeStruct((NW * L,), jnp.float32),
    ),
    scratch_types=[
        pltpu.VMEM((RPW * D,), jnp.float32),
        pltpu.VMEM((RPW * D,), jnp.float32),
        pltpu.VMEM((RPW * D,), jnp.int32),
        pltpu.VMEM((RPW * L,), jnp.float32),
        pltpu.VMEM((RPW * L,), jnp.float32),
        pltpu.VMEM((L,), jnp.float32),
        pltpu.SemaphoreType.DMA,
    ],
)
def _sc_select(x_hbm, kexp_hbm, sparse_hbm, mask_hbm, nnz_hbm, l1_hbm,
               xv, mv, bv, kv, nnzv, l1v, sem):
    wid = lax.axis_index("s") * NC + lax.axis_index("c")
    base = wid * RPW

    copies = []
    for r in range(RPW):
        copies.append(pltpu.async_copy(
            x_hbm.at[base + r], xv.at[pl.ds(r * D, D)], sem))
        copies.append(pltpu.async_copy(
            kexp_hbm.at[pl.ds((base + r) * L, L)], kv.at[pl.ds(r * L, L)], sem))
    for c in copies:
        c.wait()

    iota = lax.iota(jnp.int32, L)
    dnums = lax.GatherDimensionNumbers(
        offset_dims=(), collapsed_slice_dims=(0,), start_index_map=(0,))

    def lane_sum(v):
        for sh in (8, 4, 2, 1):
            v = v + lax.gather(v, (iota ^ sh)[:, None], dnums,
                               slice_sizes=(1,),
                               mode=lax.GatherScatterMode.PROMISE_IN_BOUNDS)
        return v

    ones = jnp.ones((L,), jnp.float32)
    zero_f = jnp.zeros((L,), jnp.float32)
    signmask = jnp.full((L,), 0x7FFFFFFF, jnp.int32)

    NACC = 8
    UNR = 16
    zero_i = jnp.zeros((L,), jnp.int32)

    def bpass(j, _):
        for t in range(UNR):
            off = j * UNR * L + t * L
            v = xv[pl.ds(off, L)]
            bv[pl.ds(off, L)] = (
                lax.bitcast_convert_type(v, jnp.int32) & signmask)
        return 0

    lax.fori_loop(0, RPW * D // (UNR * L), bpass, 0)

    l1_acc = zero_f
    outs = []
    for r in range(RPW):
        rb = r * D
        k_vec = kv[pl.ds(r * L, L)]

        def step(i, p, rb=rb):
            bitval = lax.shift_left(jnp.int32(1), jnp.int32(30) - i)
            c = p | jnp.broadcast_to(bitval, (L,))

            def inner(j, accs, rb=rb, c=c):
                accs = list(accs)
                for t in range(UNR):
                    b = bv[pl.ds(rb + j * UNR * L + t * L, L)]
                    accs[t % NACC] = accs[t % NACC] + lax.shift_right_arithmetic(b - c, 31)
                return tuple(accs)

            accs = lax.fori_loop(0, D // (UNR * L), inner, (zero_i,) * NACC)
            neg = accs[0]
            for t in range(1, NACC):
                neg = neg + accs[t]
            cnt = (-neg).astype(jnp.float32)
            return jnp.where(lane_sum(cnt) < k_vec, c, p)

        p = lax.fori_loop(0, 31, step, jnp.zeros((L,), jnp.int32))

        def mpass(j, carry, rb=rb, p=p):
            nnz0, nnz1, l10, l11 = carry
            for t in range(4):
                off = rb + j * 4 * L + t * L
                v = xv[pl.ds(off, L)]
                b = bv[pl.ds(off, L)]
                mf = jnp.where(b > p, 1.0, 0.0)
                sx = v * mf
                xv[pl.ds(off, L)] = sx
                mv[pl.ds(off, L)] = mf
                ax = jnp.abs(sx)
                if t % 2 == 0:
                    nnz0 = nnz0 + mf
                    l10 = l10 + ax
                else:
                    nnz1 = nnz1 + mf
                    l11 = l11 + ax
            return (nnz0, nnz1, l10, l11)

        nnz0, nnz1, l10, l11 = lax.fori_loop(
            0, D // (4 * L), mpass, (zero_f,) * 4)
        nnzv[pl.ds(r * L, L)] = lane_sum(nnz0 + nnz1) * (1.0 / D)
        l1_acc = l1_acc + l10 + l11

        outs.append(pltpu.async_copy(
            xv.at[pl.ds(rb, D)], sparse_hbm.at[base + r], sem))
        outs.append(pltpu.async_copy(
            mv.at[pl.ds(rb, D)], mask_hbm.at[base + r], sem))

    l1v[...] = lane_sum(l1_acc)

    outs.append(pltpu.async_copy(
        nnzv, nnz_hbm.at[pl.ds(base * L, RPW * L)], sem))
    outs.append(pltpu.async_copy(
        l1v, l1_hbm.at[pl.ds(wid * L, L)], sem))
    for c in outs:
        c.wait()


@jax.jit
def kernel(x, W1, b1, W2, b2):
    sparsity, kexp = _head(x, W1, b1, W2, b2)
    sparse_x, mask, nnz, l1p = _sc_select(x, kexp.reshape(B * L))
    actual_sparsity = nnz.reshape(B, L)[:, 0]
    l1_reg = l1p.reshape(NW, L)[:, 0].sum() * (1.0 / B)
    return (sparse_x, mask, sparsity, actual_sparsity, l1_reg)
